# baseline (speedup 1.0000x reference)
import functools

import jax
import jax.numpy as jnp
import numpy as np
from jax import lax
from jax.experimental import pallas as pl
from jax.experimental.pallas import tpu as pltpu
from jax.experimental.pallas import tpu_sc as plsc

B, N, F = 4, 4096, 64
S, K = 1024, 32
R2 = np.float32(0.4 * 0.4)
EPS = np.float32(1e-5)
P = B * S * K
CIN = 80
NW = 32
ROWS_PER_W = P // NW
CHUNK = 128
NCHUNK = ROWS_PER_W // CHUNK


FSUB, FLANE = 8, N // 8


def _fps_body(pts_ref, out_ref):
    arr = pts_ref[...]
    iota = (lax.broadcasted_iota(jnp.int32, (B, FSUB, FLANE), 1) * FLANE
            + lax.broadcasted_iota(jnp.int32, (B, FSUB, FLANE), 2))
    iota4 = iota[:, None]

    def body(i, carry):
        dists, far = carry
        sel4 = iota4 == far[:, None]
        cent = jnp.sum(jnp.where(sel4, arr, 0.0), axis=(2, 3),
                       keepdims=True)
        out_ref[pl.ds(i, 1), :, :] = cent.reshape(1, B, 3)
        diff = arr - cent
        sq = diff * diff
        d = (sq[:, 0] + sq[:, 1]) + sq[:, 2]
        dists = jnp.minimum(dists, d)
        m = jnp.max(dists, axis=(1, 2), keepdims=True)
        cand = jnp.where(dists == m, iota, N)
        far = jnp.min(cand, axis=(1, 2), keepdims=True)
        return dists, far

    dists0 = jnp.full((B, FSUB, FLANE), 1e10, dtype=jnp.float32)
    far0 = jnp.zeros((B, 1, 1), dtype=jnp.int32)
    lax.fori_loop(0, S, body, (dists0, far0))


def _fps(pts):
    return pl.pallas_call(
        _fps_body,
        out_shape=jax.ShapeDtypeStruct((S, B, 3), jnp.float32),
    )(pts.reshape(B, 3, FSUB, FLANE))


S_T = 128
NW32 = N // 32


def _pack_matrix():
    p = np.zeros((N, 3 * NW32), np.float32)
    n = np.arange(N)
    w, j = n // 32, n % 32
    lo = j < 16
    p[n[lo], w[lo]] = (2.0 ** j[lo])
    p[n[~lo], NW32 + w[~lo]] = (2.0 ** (j[~lo] - 16))
    p[n, 2 * NW32 + w] = 1.0
    return jnp.asarray(p, dtype=jnp.bfloat16)


def _bq_body(cents_ref, pts_ref, pack_ref, out_ref):
    c = cents_ref[0]
    p = pts_ref[0]
    x = p[0:1, :]
    y = p[1:2, :]
    z = p[2:3, :]
    cx = c[:, 0:1]
    cy = c[:, 1:2]
    cz = c[:, 2:3]
    an = (cx * cx + cy * cy) + cz * cz
    bn = (x * x + y * y) + z * z
    rnd = lambda v: v.astype(jnp.bfloat16).astype(jnp.float32)
    cross = (rnd(cx) * rnd(x) + rnd(cy) * rnd(y)) + rnd(cz) * rnd(z)
    d2 = (an + bn) - 2.0 * cross
    mbf = (d2 <= R2).astype(jnp.bfloat16)
    packed = jnp.dot(mbf, pack_ref[...],
                     preferred_element_type=jnp.float32)
    lo = packed[:, 0:NW32]
    hi = packed[:, NW32:2 * NW32]
    pc = packed[:, 2 * NW32:3 * NW32]
    c = pc
    sh = 1
    while sh < NW32:
        c = c + jnp.concatenate(
            [jnp.zeros((S_T, sh), jnp.float32), c[:, :NW32 - sh]], axis=1)
        sh *= 2
    kk = lax.broadcasted_iota(jnp.int32, (S_T, K, NW32), 1).astype(jnp.float32)
    ind = (c[:, None, :] <= kk).astype(jnp.float32)
    wk = jnp.sum(ind, axis=2)
    basek = jnp.sum(ind * pc[:, None, :], axis=2)
    wiota = lax.broadcasted_iota(
        jnp.int32, (S_T, K, NW32), 2).astype(jnp.float32)
    sel = (wiota == wk[:, :, None]).astype(jnp.float32)
    wlo = jnp.sum(sel * lo[:, None, :], axis=2).astype(jnp.int32)
    whi = jnp.sum(sel * hi[:, None, :], axis=2).astype(jnp.int32)
    kvec = lax.broadcasted_iota(jnp.int32, (S_T, K), 1)
    r = (kvec - basek.astype(jnp.int32))
    cnt = jnp.zeros((S_T, K), jnp.int32)
    pos = jnp.zeros((S_T, K), jnp.int32)
    for j in range(32):
        word = wlo if j < 16 else whi
        bit = (word >> (j % 16)) & 1
        pos = jnp.where((bit == 1) & (cnt == r), j, pos)
        cnt = cnt + bit
    idx = wk.astype(jnp.int32) * 32 + pos
    t = c[:, NW32 - 1:NW32].astype(jnp.int32)
    idx = jnp.where(kvec < t, idx, idx[:, 0:1])
    b = pl.program_id(0)
    out_ref[0] = idx + b * N


def _ball_query(cents, pts):
    return pl.pallas_call(
        _bq_body,
        grid=(B, S // S_T),
        in_specs=[
            pl.BlockSpec((1, S_T, 3), lambda b, sb: (b, sb, 0)),
            pl.BlockSpec((1, 3, N), lambda b, sb: (b, 0, 0)),
            pl.BlockSpec((N, 3 * NW32), lambda b, sb: (0, 0)),
        ],
        out_specs=pl.BlockSpec((1, S_T, K), lambda b, sb: (b, sb, 0)),
        out_shape=jax.ShapeDtypeStruct((B, S, K), jnp.int32),
    )(cents, pts, _pack_matrix())


def _sc_gather_body(table_hbm, idx_hbm, out_hbm, idx_v, rows_a, rows_b,
                    gsem, ssem_a, ssem_b):
    wid = lax.axis_index("c") * 16 + lax.axis_index("s")
    base = wid * ROWS_PER_W
    pltpu.sync_copy(idx_hbm.at[wid], idx_v)

    def body(jj, carry):
        j0 = jj * 2
        pltpu.async_copy(table_hbm.at[idx_v.at[j0]], rows_a, gsem).wait()
        sa = pltpu.async_copy(
            rows_a, out_hbm.at[pl.ds(base + j0 * CHUNK, CHUNK)], ssem_a)
        pltpu.async_copy(table_hbm.at[idx_v.at[j0 + 1]], rows_b, gsem).wait()
        sb = pltpu.async_copy(
            rows_b, out_hbm.at[pl.ds(base + (j0 + 1) * CHUNK, CHUNK)], ssem_b)
        sa.wait()
        sb.wait()
        return carry

    lax.fori_loop(0, NCHUNK // 2, body, 0)


def _sc_gather(table, gidx):
    mesh = plsc.VectorSubcoreMesh(core_axis_name="c", subcore_axis_name="s")
    return pl.kernel(
        _sc_gather_body,
        out_type=jax.ShapeDtypeStruct((P, CIN), jnp.float32),
        mesh=mesh,
        scratch_types=[
            pltpu.VMEM((NCHUNK, CHUNK), jnp.int32),
            pltpu.VMEM((CHUNK, CIN), jnp.float32),
            pltpu.VMEM((CHUNK, CIN), jnp.float32),
            pltpu.SemaphoreType.DMA,
            pltpu.SemaphoreType.DMA,
            pltpu.SemaphoreType.DMA,
        ],
        compiler_params=pltpu.CompilerParams(use_tc_tiling_on_sc=False),
    )(table, gidx)


P_T = 2048
G_T = P_T // K
NSTEP = P // P_T
INV_P = np.float32(1.0 / P)


def _mlp0_body(x_ref, c_ref, w_ref, b_ref, h_ref, st_ref):
    w = w_ref[...]
    h = jnp.dot(x_ref[...], w, preferred_element_type=jnp.float32)
    c = c_ref[...]
    corr = (c[:, 0:1] * w[0:1, :] + c[:, 1:2] * w[1:2, :]
            + c[:, 2:3] * w[2:3, :])
    corr_full = jnp.broadcast_to(
        corr[:, None, :], (G_T, K, corr.shape[-1])).reshape(P_T, -1)
    h = (h + b_ref[...]) - corr_full
    h_ref[...] = h

    @pl.when(pl.program_id(0) == 0)
    def _():
        st_ref[...] = jnp.zeros_like(st_ref)

    st_ref[0:1, :] += jnp.sum(h, axis=0, keepdims=True)
    st_ref[1:2, :] += jnp.sum(h * h, axis=0, keepdims=True)


def _mlp0(x, cents_flat, w0t, b0):
    cout = w0t.shape[1]
    return pl.pallas_call(
        _mlp0_body,
        grid=(NSTEP,),
        in_specs=[
            pl.BlockSpec((P_T, CIN), lambda i: (i, 0)),
            pl.BlockSpec((G_T, 3), lambda i: (i, 0)),
            pl.BlockSpec((CIN, cout), lambda i: (0, 0)),
            pl.BlockSpec((1, cout), lambda i: (0, 0)),
        ],
        out_specs=[
            pl.BlockSpec((P_T, cout), lambda i: (i, 0)),
            pl.BlockSpec((8, cout), lambda i: (0, 0)),
        ],
        out_shape=[
            jax.ShapeDtypeStruct((P, cout), jnp.float32),
            jax.ShapeDtypeStruct((8, cout), jnp.float32),
        ],
    )(x, cents_flat, w0t, b0)


def _norm_act(h, st_ref, g_ref, be_ref):
    mean = st_ref[0:1, :] * INV_P
    var = st_ref[1:2, :] * INV_P - mean * mean
    hn = (h - mean) / jnp.sqrt(var + EPS) * g_ref[...] + be_ref[...]
    return jnp.where(hn >= 0, hn, 0.2 * hn)


def _mlp_mid_body(h_ref, st_in_ref, g_ref, be_ref, w_ref, b_ref,
                  o_ref, st_ref):
    act = _norm_act(h_ref[...], st_in_ref, g_ref, be_ref)
    h = jnp.dot(act, w_ref[...], preferred_element_type=jnp.float32)
    h = h + b_ref[...]
    o_ref[...] = h

    @pl.when(pl.program_id(0) == 0)
    def _():
        st_ref[...] = jnp.zeros_like(st_ref)

    st_ref[0:1, :] += jnp.sum(h, axis=0, keepdims=True)
    st_ref[1:2, :] += jnp.sum(h * h, axis=0, keepdims=True)


def _mlp_mid(h, st, g, be, wt, b):
    cin, cout = wt.shape
    return pl.pallas_call(
        _mlp_mid_body,
        grid=(NSTEP,),
        in_specs=[
            pl.BlockSpec((P_T, cin), lambda i: (i, 0)),
            pl.BlockSpec((8, cin), lambda i: (0, 0)),
            pl.BlockSpec((1, cin), lambda i: (0, 0)),
            pl.BlockSpec((1, cin), lambda i: (0, 0)),
            pl.BlockSpec((cin, cout), lambda i: (0, 0)),
            pl.BlockSpec((1, cout), lambda i: (0, 0)),
        ],
        out_specs=[
            pl.BlockSpec((P_T, cout), lambda i: (i, 0)),
            pl.BlockSpec((8, cout), lambda i: (0, 0)),
        ],
        out_shape=[
            jax.ShapeDtypeStruct((P, cout), jnp.float32),
            jax.ShapeDtypeStruct((8, cout), jnp.float32),
        ],
    )(h, st, g, be, wt, b)


def _fin_body(h_ref, st_in_ref, g_ref, be_ref, o_ref):
    act = _norm_act(h_ref[...], st_in_ref, g_ref, be_ref)
    cout = act.shape[-1]
    o_ref[...] = jnp.max(act.reshape(G_T, K, cout), axis=1)


def _mlp_fin(h, st, g, be):
    cin = h.shape[1]
    return pl.pallas_call(
        _fin_body,
        grid=(NSTEP,),
        in_specs=[
            pl.BlockSpec((P_T, cin), lambda i: (i, 0)),
            pl.BlockSpec((8, cin), lambda i: (0, 0)),
            pl.BlockSpec((1, cin), lambda i: (0, 0)),
            pl.BlockSpec((1, cin), lambda i: (0, 0)),
        ],
        out_specs=pl.BlockSpec((G_T, cin), lambda i: (i, 0)),
        out_shape=jax.ShapeDtypeStruct((B * S, cin), jnp.float32),
    )(h, st, g, be)


def kernel(features, points, W0, b0, gamma0, beta0, W1, b1, gamma1, beta1,
           W2, b2, gamma2, beta2):
    pts = points.astype(jnp.float32)

    cent_rows = _fps(pts)
    cents = jnp.transpose(cent_rows, (1, 0, 2))
    new_xyz = jnp.transpose(cent_rows, (1, 2, 0))

    gidx = _ball_query(cents, pts)

    table = jnp.concatenate(
        [jnp.transpose(pts, (0, 2, 1)),
         jnp.transpose(features, (0, 2, 1)),
         jnp.zeros((B, N, CIN - 3 - F), jnp.float32)],
        axis=-1).reshape(B * N, CIN)
    x = _sc_gather(table, gidx.reshape(NW, NCHUNK, CHUNK))

    w0t = jnp.pad(W0, ((0, 0), (0, CIN - 3 - F))).T
    cents_flat = cents.reshape(B * S, 3)
    h0, st0 = _mlp0(x, cents_flat, w0t, b0.reshape(1, -1))
    h1, st1 = _mlp_mid(h0, st0, gamma0.reshape(1, -1), beta0.reshape(1, -1),
                       W1.T, b1.reshape(1, -1))
    h2, st2 = _mlp_mid(h1, st1, gamma1.reshape(1, -1), beta1.reshape(1, -1),
                       W2.T, b2.reshape(1, -1))
    pooled = _mlp_fin(h2, st2, gamma2.reshape(1, -1), beta2.reshape(1, -1))

    new_features = jnp.transpose(pooled.reshape(B, S, -1), (0, 2, 1))
    return (new_features, new_xyz)
